# Initial kernel scaffold; baseline (speedup 1.0000x reference)
import jax
import jax.numpy as jnp
from jax.experimental import pallas as pl


def kernel(x, edge_index, weight_in, weight_out, conv_weight, w_ih, w_hh, b_ih, b_hh):
    raise NotImplementedError("write your pallas kernel here")


# speedup vs baseline: 4.4276x; 4.4276x over previous
import functools

import jax
import jax.numpy as jnp
from jax import lax
from jax.experimental import pallas as pl
from jax.experimental.pallas import tpu as pltpu
from jax.experimental.pallas import tpu_sc as plsc

_NC = 2
_NS = 16
_NW = _NC * _NS
_LANES = 128


def _make_sc_segment_sum(n_rows, feat, j_chunks, agg_rows):
  rows_per_tile = agg_rows // _NS
  n_zero_copies = rows_per_tile // _LANES
  mesh = plsc.VectorSubcoreMesh(core_axis_name="c", subcore_axis_name="s")

  @functools.partial(
      pl.kernel,
      out_type=jax.ShapeDtypeStruct((_NC, agg_rows, feat), jnp.float32),
      mesh=mesh,
      scratch_types=[
          pltpu.VMEM((j_chunks, _LANES), jnp.int32),
          pltpu.VMEM((j_chunks, _LANES), jnp.int32),
          pltpu.VMEM((_LANES, feat), jnp.float32),
          pltpu.VMEM_SHARED((agg_rows, feat), jnp.float32),
          pltpu.SemaphoreType.DMA,
      ],
  )
  def sc_segment_sum(m_hbm, src_hbm, dst_hbm, out_hbm, src_v, dst_v, rows_v,
                     agg_sh, sem):
    c = lax.axis_index("c")
    s = lax.axis_index("s")
    wid = c * _NS + s

    def _zrow(i, carry):
      def _zcol(j, carry2):
        rows_v[i, pl.ds(j * 16, 16)] = jnp.zeros((16,), jnp.float32)
        return carry2
      return lax.fori_loop(0, feat // 16, _zcol, carry)
    lax.fori_loop(0, _LANES, _zrow, 0)

    base = s * rows_per_tile
    for t in range(n_zero_copies):
      pltpu.sync_copy(rows_v, agg_sh.at[pl.ds(base + t * _LANES, _LANES)])

    pltpu.sync_copy(src_hbm.at[wid], src_v)
    pltpu.sync_copy(dst_hbm.at[wid], dst_v)
    plsc.subcore_barrier()

    def _body(j, carry):
      pltpu.async_copy(m_hbm.at[src_v.at[j]], rows_v, sem).wait()
      pltpu.sync_copy(rows_v, agg_sh.at[dst_v.at[j]], add=True)
      return carry
    lax.fori_loop(0, j_chunks, _body, 0)
    plsc.subcore_barrier()

    for t in range(n_zero_copies):
      sl = pl.ds(base + t * _LANES, _LANES)
      pltpu.sync_copy(agg_sh.at[sl], out_hbm.at[c].at[sl])

  return sc_segment_sum


def _tc_matmul(h, w, blk):
  n, f = h.shape
  c2 = w.shape[1]

  def body(h_ref, w_ref, o_ref):
    o_ref[...] = jnp.dot(h_ref[...], w_ref[...],
                         preferred_element_type=jnp.float32)

  return pl.pallas_call(
      body,
      grid=(n // blk,),
      in_specs=[
          pl.BlockSpec((blk, f), lambda i: (i, 0)),
          pl.BlockSpec((f, c2), lambda i: (0, 0)),
      ],
      out_specs=pl.BlockSpec((blk, c2), lambda i: (i, 0)),
      out_shape=jax.ShapeDtypeStruct((n, c2), jnp.float32),
  )(h, w)


def _tc_gru(p0, p1, h, wih_t, whh_t, bih, bhh, blk):
  n, c = h.shape

  def body(p0_ref, p1_ref, h_ref, wih_ref, whh_ref, bih_ref, bhh_ref, o_ref):
    agg = p0_ref[...] + p1_ref[...]
    hh = h_ref[...]
    gi = jnp.dot(agg, wih_ref[...],
                 preferred_element_type=jnp.float32) + bih_ref[...]
    gh = jnp.dot(hh, whh_ref[...],
                 preferred_element_type=jnp.float32) + bhh_ref[...]
    r = jax.nn.sigmoid(gi[:, :c] + gh[:, :c])
    z = jax.nn.sigmoid(gi[:, c:2 * c] + gh[:, c:2 * c])
    nn = jnp.tanh(gi[:, 2 * c:] + r * gh[:, 2 * c:])
    o_ref[...] = (1.0 - z) * nn + z * hh

  return pl.pallas_call(
      body,
      grid=(n // blk,),
      in_specs=[
          pl.BlockSpec((blk, c), lambda i: (i, 0)),
          pl.BlockSpec((blk, c), lambda i: (i, 0)),
          pl.BlockSpec((blk, c), lambda i: (i, 0)),
          pl.BlockSpec((c, 3 * c), lambda i: (0, 0)),
          pl.BlockSpec((c, 3 * c), lambda i: (0, 0)),
          pl.BlockSpec((1, 3 * c), lambda i: (0, 0)),
          pl.BlockSpec((1, 3 * c), lambda i: (0, 0)),
      ],
      out_specs=pl.BlockSpec((blk, c), lambda i: (i, 0)),
      out_shape=jax.ShapeDtypeStruct((n, c), jnp.float32),
  )(p0, p1, h, wih_t, whh_t, bih, bhh)


def _tc_out_logsoftmax(h, w, blk):
  n, c = h.shape
  k = w.shape[1]

  def body(h_ref, w_ref, o_ref):
    logits = jnp.dot(h_ref[...], w_ref[...],
                     preferred_element_type=jnp.float32)
    mx = jnp.max(logits, axis=1, keepdims=True)
    sh = logits - mx
    lse = jnp.log(jnp.sum(jnp.exp(sh), axis=1, keepdims=True))
    o_ref[...] = sh - lse

  return pl.pallas_call(
      body,
      grid=(n // blk,),
      in_specs=[
          pl.BlockSpec((blk, c), lambda i: (i, 0)),
          pl.BlockSpec((c, k), lambda i: (0, 0)),
      ],
      out_specs=pl.BlockSpec((blk, k), lambda i: (i, 0)),
      out_shape=jax.ShapeDtypeStruct((n, k), jnp.float32),
  )(h, w)


def kernel(x, edge_index, weight_in, weight_out, conv_weight, w_ih, w_hh,
           b_ih, b_hh):
  n, f = x.shape
  num_layers = conv_weight.shape[0]
  e = edge_index.shape[1]

  batch = _NW * _LANES
  ep = ((e + batch - 1) // batch) * batch
  pad = ep - e
  src = jnp.concatenate([edge_index[0], jnp.zeros((pad,), jnp.int32)])
  dst = jnp.concatenate([edge_index[1], jnp.full((pad,), n, jnp.int32)])
  j_chunks = ep // batch
  src3 = src.reshape(_NW, j_chunks, _LANES)
  dst3 = dst.reshape(_NW, j_chunks, _LANES)

  tile_quant = _NS * _LANES
  agg_rows = ((n + 1 + tile_quant - 1) // tile_quant) * tile_quant

  sc_segment_sum = _make_sc_segment_sum(n, f, j_chunks, agg_rows)

  blk = 2000
  wih_t = w_ih.T
  whh_t = w_hh.T
  bih2 = b_ih.reshape(1, -1)
  bhh2 = b_hh.reshape(1, -1)

  h = _tc_matmul(x, weight_in, blk)
  for i in range(num_layers):
    m = _tc_matmul(h, conv_weight[i], blk)
    parts = sc_segment_sum(m, src3, dst3)
    h = _tc_gru(parts[0, :n], parts[1, :n], h, wih_t, whh_t, bih2, bhh2, blk)
  return _tc_out_logsoftmax(h, weight_out, blk)
